# baseline (speedup 1.0000x reference)
import functools

import jax
import jax.numpy as jnp
from jax import lax
from jax.experimental import pallas as pl
from jax.experimental.pallas import tpu as pltpu
from jax.experimental.pallas import tpu_sc as plsc

B = 4096
L = 200
WORD_DIM = 50
OUT_DIM = 60
PAD_DIM = 64
TOK = B * L

_info = plsc.get_sparse_core_info()
NC, NS, LANES = _info.num_cores, _info.num_subcores, _info.num_lanes
NW = NC * NS

PER_W = TOK // NW
CHUNK = 256
NCHUNK = PER_W // CHUNK
IDX_PER_DMA = 128
NDMA = CHUNK // IDX_PER_DMA


def _sc_embed(word_pad, widx, p1idx, p2idx, p1t, p2t):
    mesh = plsc.VectorSubcoreMesh(core_axis_name="c", subcore_axis_name="s")

    @functools.partial(
        pl.kernel,
        mesh=mesh,
        out_type=jax.ShapeDtypeStruct((TOK, OUT_DIM), jnp.float32),
        compiler_params=pltpu.CompilerParams(
            needs_layout_passes=False, use_tc_tiling_on_sc=False),
        scratch_types=[
            pltpu.VMEM((CHUNK,), jnp.int32),
            pltpu.VMEM((CHUNK,), jnp.int32),
            pltpu.VMEM((CHUNK,), jnp.int32),
            pltpu.VMEM((CHUNK,), jnp.int32),
            pltpu.VMEM((CHUNK,), jnp.int32),
            pltpu.VMEM((CHUNK,), jnp.int32),
            pltpu.VMEM((CHUNK, PAD_DIM), jnp.float32),
            pltpu.VMEM((CHUNK, PAD_DIM), jnp.float32),
            pltpu.VMEM((CHUNK, OUT_DIM), jnp.float32),
            pltpu.VMEM((CHUNK, OUT_DIM), jnp.float32),
            pltpu.VMEM((2 * L * 5,), jnp.float32),
            pltpu.VMEM((2 * L * 5,), jnp.float32),
            pltpu.SemaphoreType.DMA,
            pltpu.SemaphoreType.DMA,
            pltpu.SemaphoreType.DMA,
            pltpu.SemaphoreType.DMA,
        ],
    )
    def k(word_hbm, widx_hbm, p1idx_hbm, p2idx_hbm, p1t_hbm, p2t_hbm,
          out_hbm, widxA, widxB, p1iA, p1iB, p2iA, p2iB,
          rowsA, rowsB, outA, outB, p1_v, p2_v,
          gsemA, gsemB, wsemA, wsemB):
        wid = lax.axis_index("s") * NC + lax.axis_index("c")
        base_w = wid * PER_W
        pltpu.sync_copy(p1t_hbm, p1_v)
        pltpu.sync_copy(p2t_hbm, p2_v)

        def stage_idx(ci, widx_v, p1i_v, p2i_v):
            base = base_w + ci * CHUNK
            pltpu.sync_copy(widx_hbm.at[pl.ds(base, CHUNK)], widx_v)
            pltpu.sync_copy(p1idx_hbm.at[pl.ds(base, CHUNK)], p1i_v)
            pltpu.sync_copy(p2idx_hbm.at[pl.ds(base, CHUNK)], p2i_v)

        def fire_gather(widx_v, rows, gsem):
            for di in range(NDMA):
                pltpu.async_copy(
                    word_hbm.at[widx_v.at[pl.ds(di * IDX_PER_DMA,
                                                IDX_PER_DMA)]],
                    rows.at[pl.ds(di * IDX_PER_DMA, IDX_PER_DMA), :],
                    gsem)

        def wait_gather(rows, gsem):
            pltpu.make_async_copy(
                word_hbm.at[pl.ds(0, CHUNK), :], rows, gsem).wait()

        def fire_wb(out_v, ci, wsem):
            base = base_w + ci * CHUNK
            pltpu.async_copy(out_v, out_hbm.at[pl.ds(base, CHUNK), :], wsem)

        def wait_wb(out_v, wsem):
            pltpu.make_async_copy(
                out_v, out_hbm.at[pl.ds(0, CHUNK), :], wsem).wait()

        def repack(rows, out_v, p1i_v, p2i_v):
            def grp(g, carry):
                tb = g * LANES
                t16 = lax.iota(jnp.int32, LANES) + tb
                p1i = p1i_v[pl.ds(tb, LANES)] * 5
                p2i = p2i_v[pl.ds(tb, LANES)] * 5
                for kk in range(LANES):
                    t = tb + kk
                    out_v[t, pl.ds(0, 16)] = rows[t, pl.ds(0, 16)]
                    out_v[t, pl.ds(16, 16)] = rows[t, pl.ds(16, 16)]
                    out_v[t, pl.ds(32, 16)] = rows[t, pl.ds(32, 16)]
                    out_v[t, pl.ds(44, 16)] = rows[t, pl.ds(44, 16)]
                for j in range(5):
                    v1 = plsc.load_gather(p1_v, [p1i + j])
                    plsc.store_scatter(
                        out_v,
                        [t16, jnp.full((LANES,), WORD_DIM + j, jnp.int32)],
                        v1)
                    v2 = plsc.load_gather(p2_v, [p2i + j])
                    plsc.store_scatter(
                        out_v,
                        [t16, jnp.full((LANES,), WORD_DIM + 5 + j,
                                       jnp.int32)],
                        v2)
                return carry

            lax.fori_loop(0, CHUNK // LANES, grp, 0)

        stage_idx(0, widxA, p1iA, p2iA)
        fire_gather(widxA, rowsA, gsemA)

        def body(i2, carry):
            iA = 2 * i2
            iB = iA + 1
            wait_gather(rowsA, gsemA)
            stage_idx(iB, widxB, p1iB, p2iB)
            fire_gather(widxB, rowsB, gsemB)

            @pl.when(iA >= 2)
            def _():
                wait_wb(outA, wsemA)

            repack(rowsA, outA, p1iA, p2iA)
            fire_wb(outA, iA, wsemA)

            wait_gather(rowsB, gsemB)

            @pl.when(iB + 1 < NCHUNK)
            def _():
                stage_idx(iB + 1, widxA, p1iA, p2iA)
                fire_gather(widxA, rowsA, gsemA)

            @pl.when(iB >= 2)
            def _():
                wait_wb(outB, wsemB)

            repack(rowsB, outB, p1iB, p2iB)
            fire_wb(outB, iB, wsemB)
            return carry

        lax.fori_loop(0, NCHUNK // 2, body, 0)
        wait_wb(outA, wsemA)
        wait_wb(outB, wsemB)

    return k(word_pad, widx, p1idx, p2idx, p1t, p2t)


def kernel(word, pos1, pos2, word_table, pos1_table, pos2_table):
    word_pad = jnp.pad(word_table, ((0, 0), (0, PAD_DIM - WORD_DIM)))
    out2d = _sc_embed(
        word_pad,
        word.reshape(-1),
        pos1.reshape(-1),
        pos2.reshape(-1),
        pos1_table.reshape(-1),
        pos2_table.reshape(-1),
    )
    return out2d.reshape(B, L, OUT_DIM)


# speedup vs baseline: 1.3346x; 1.3346x over previous
import functools

import jax
import jax.numpy as jnp
from jax import lax
from jax.experimental import pallas as pl
from jax.experimental.pallas import tpu as pltpu
from jax.experimental.pallas import tpu_sc as plsc

B = 4096
L = 200
WORD_DIM = 50
OUT_DIM = 60
PAD_DIM = 64
TOK = B * L

_info = plsc.get_sparse_core_info()
NC, NS, LANES = _info.num_cores, _info.num_subcores, _info.num_lanes
NW = NC * NS

PER_W = TOK // NW
CHUNK = 256
NBUF = 4
NCHUNK = PER_W // CHUNK
IDX_PER_DMA = 128
NDMA = CHUNK // IDX_PER_DMA


def _sc_embed(word_pad, widx, p1idx, p2idx, p1t, p2t):
    mesh = plsc.VectorSubcoreMesh(core_axis_name="c", subcore_axis_name="s")

    scratch = (
        [pltpu.VMEM((CHUNK,), jnp.int32) for _ in range(NBUF)]
        + [pltpu.VMEM((CHUNK,), jnp.int32) for _ in range(NBUF)]
        + [pltpu.VMEM((CHUNK,), jnp.int32) for _ in range(NBUF)]
        + [pltpu.VMEM((CHUNK, PAD_DIM), jnp.float32)
           for _ in range(NBUF)]
        + [pltpu.VMEM((2 * L * 5,), jnp.float32)] * 2
        + [pltpu.SemaphoreType.DMA for _ in range(NBUF)]
        + [pltpu.SemaphoreType.DMA for _ in range(NBUF)]
    )

    @functools.partial(
        pl.kernel,
        mesh=mesh,
        out_type=jax.ShapeDtypeStruct((TOK, PAD_DIM), jnp.float32),
        compiler_params=pltpu.CompilerParams(
            needs_layout_passes=False, use_tc_tiling_on_sc=False),
        scratch_types=scratch,
    )
    def k(word_hbm, widx_hbm, p1idx_hbm, p2idx_hbm, p1t_hbm, p2t_hbm,
          out_hbm, *refs):
        widx_v = refs[0:NBUF]
        p1i_v = refs[NBUF:2 * NBUF]
        p2i_v = refs[2 * NBUF:3 * NBUF]
        rows_v = refs[3 * NBUF:4 * NBUF]
        p1_v, p2_v = refs[4 * NBUF], refs[4 * NBUF + 1]
        gsem = refs[4 * NBUF + 2:5 * NBUF + 2]
        wsem = refs[5 * NBUF + 2:6 * NBUF + 2]

        wid = lax.axis_index("s") * NC + lax.axis_index("c")
        base_w = wid * PER_W
        pltpu.sync_copy(p1t_hbm, p1_v)
        pltpu.sync_copy(p2t_hbm, p2_v)

        def stage_and_fire(ci, p):
            base = base_w + ci * CHUNK
            pltpu.sync_copy(widx_hbm.at[pl.ds(base, CHUNK)], widx_v[p])
            pltpu.sync_copy(p1idx_hbm.at[pl.ds(base, CHUNK)], p1i_v[p])
            pltpu.sync_copy(p2idx_hbm.at[pl.ds(base, CHUNK)], p2i_v[p])
            for di in range(NDMA):
                pltpu.async_copy(
                    word_hbm.at[widx_v[p].at[pl.ds(di * IDX_PER_DMA,
                                                   IDX_PER_DMA)]],
                    rows_v[p].at[pl.ds(di * IDX_PER_DMA, IDX_PER_DMA), :],
                    gsem[p])

        def wait_gather(p):
            pltpu.make_async_copy(
                word_hbm.at[pl.ds(0, CHUNK), :], rows_v[p], gsem[p]).wait()

        def wait_wb(p):
            pltpu.make_async_copy(
                rows_v[p], out_hbm.at[pl.ds(0, CHUNK), :], wsem[p]).wait()

        def pos_fill(p):
            def grp(g, carry):
                tb = g * LANES
                t16 = lax.iota(jnp.int32, LANES) + tb
                p1i = p1i_v[p][pl.ds(tb, LANES)] * 5
                p2i = p2i_v[p][pl.ds(tb, LANES)] * 5
                for j in range(5):
                    v1 = plsc.load_gather(p1_v, [p1i + j])
                    plsc.store_scatter(
                        rows_v[p],
                        [t16, jnp.full((LANES,), WORD_DIM + j, jnp.int32)],
                        v1)
                    v2 = plsc.load_gather(p2_v, [p2i + j])
                    plsc.store_scatter(
                        rows_v[p],
                        [t16, jnp.full((LANES,), WORD_DIM + 5 + j,
                                       jnp.int32)],
                        v2)
                return carry

            lax.fori_loop(0, CHUNK // LANES, grp, 0)

        stage_and_fire(0, 0)

        def body(i4, carry):
            for p in range(NBUF):
                j = NBUF * i4 + p
                q = (p + 1) % NBUF
                wait_gather(p)

                @pl.when(j + 1 < NCHUNK)
                def _():
                    @pl.when(j >= NBUF - 1)
                    def _():
                        wait_wb(q)

                    stage_and_fire(j + 1, q)

                pos_fill(p)
                pltpu.async_copy(
                    rows_v[p],
                    out_hbm.at[pl.ds(base_w + j * CHUNK, CHUNK), :],
                    wsem[p])
            return carry

        lax.fori_loop(0, NCHUNK // NBUF, body, 0)
        for p in range(NBUF):
            wait_wb(p)

    return k(word_pad, widx, p1idx, p2idx, p1t, p2t)


def kernel(word, pos1, pos2, word_table, pos1_table, pos2_table):
    word_pad = jnp.pad(word_table, ((0, 0), (0, PAD_DIM - WORD_DIM)))
    out_pad = _sc_embed(
        word_pad,
        word.reshape(-1),
        pos1.reshape(-1),
        pos2.reshape(-1),
        pos1_table.reshape(-1),
        pos2_table.reshape(-1),
    )
    return out_pad[:, :OUT_DIM].reshape(B, L, OUT_DIM)


# speedup vs baseline: 1.3910x; 1.0423x over previous
import functools

import jax
import jax.numpy as jnp
from jax import lax
from jax.experimental import pallas as pl
from jax.experimental.pallas import tpu as pltpu
from jax.experimental.pallas import tpu_sc as plsc

B = 4096
L = 200
WORD_DIM = 50
OUT_DIM = 60
PAD_DIM = 64
TOK = B * L

_info = plsc.get_sparse_core_info()
NC, NS, LANES = _info.num_cores, _info.num_subcores, _info.num_lanes
NW = NC * NS

PER_W = TOK // NW
CHUNK = 256
NBUF = 4
NCHUNK = PER_W // CHUNK
IDX_PER_DMA = 128
NDMA = CHUNK // IDX_PER_DMA


def _sc_embed(word_pad, widx, p1idx, p2idx, p1t, p2t):
    mesh = plsc.VectorSubcoreMesh(core_axis_name="c", subcore_axis_name="s")

    scratch = (
        [pltpu.VMEM((CHUNK,), jnp.int32) for _ in range(NBUF)]
        + [pltpu.VMEM((CHUNK,), jnp.int32) for _ in range(NBUF)]
        + [pltpu.VMEM((CHUNK,), jnp.int32) for _ in range(NBUF)]
        + [pltpu.VMEM((CHUNK, PAD_DIM), jnp.float32)
           for _ in range(NBUF)]
        + [pltpu.VMEM((2 * 2 * L * 5,), jnp.float32)]
        + [pltpu.SemaphoreType.DMA for _ in range(NBUF)]
        + [pltpu.SemaphoreType.DMA for _ in range(NBUF)]
    )

    @functools.partial(
        pl.kernel,
        mesh=mesh,
        out_type=jax.ShapeDtypeStruct((TOK, PAD_DIM), jnp.float32),
        compiler_params=pltpu.CompilerParams(
            needs_layout_passes=False, use_tc_tiling_on_sc=False),
        scratch_types=scratch,
    )
    def k(word_hbm, widx_hbm, p1idx_hbm, p2idx_hbm, p1t_hbm, p2t_hbm,
          out_hbm, *refs):
        widx_v = refs[0:NBUF]
        p1i_v = refs[NBUF:2 * NBUF]
        p2i_v = refs[2 * NBUF:3 * NBUF]
        rows_v = refs[3 * NBUF:4 * NBUF]
        pcat_v = refs[4 * NBUF]
        gsem = refs[4 * NBUF + 1:5 * NBUF + 1]
        wsem = refs[5 * NBUF + 1:6 * NBUF + 1]

        wid = lax.axis_index("s") * NC + lax.axis_index("c")
        base_w = wid * PER_W
        pltpu.sync_copy(p1t_hbm, pcat_v.at[pl.ds(0, 2 * L * 5)])
        pltpu.sync_copy(p2t_hbm, pcat_v.at[pl.ds(2 * L * 5, 2 * L * 5)])

        def stage_and_fire(ci, p):
            base = base_w + ci * CHUNK
            pltpu.sync_copy(widx_hbm.at[pl.ds(base, CHUNK)], widx_v[p])
            pltpu.sync_copy(p1idx_hbm.at[pl.ds(base, CHUNK)], p1i_v[p])
            pltpu.sync_copy(p2idx_hbm.at[pl.ds(base, CHUNK)], p2i_v[p])
            for di in range(NDMA):
                pltpu.async_copy(
                    word_hbm.at[widx_v[p].at[pl.ds(di * IDX_PER_DMA,
                                                   IDX_PER_DMA)]],
                    rows_v[p].at[pl.ds(di * IDX_PER_DMA, IDX_PER_DMA), :],
                    gsem[p])

        def wait_gather(p):
            pltpu.make_async_copy(
                word_hbm.at[pl.ds(0, CHUNK), :], rows_v[p], gsem[p]).wait()

        def wait_wb(p):
            pltpu.make_async_copy(
                rows_v[p], out_hbm.at[pl.ds(0, CHUNK), :], wsem[p]).wait()

        def pos_fill(p):
            iota = lax.iota(jnp.int32, LANES)
            diags = []
            for s in range(10):
                jj = iota + s
                jj = jnp.where(jj >= 10, jj - 10, jj)
                jj = jnp.where(jj >= 10, jj - 10, jj)
                diags.append(jj)

            def grp(g, carry):
                tb = g * LANES
                t16 = iota + tb
                p1i = p1i_v[p][pl.ds(tb, LANES)] * 5
                p2i = p2i_v[p][pl.ds(tb, LANES)] * 5 + (2 * L * 5 - 5)
                for s in range(10):
                    jj = diags[s]
                    src = jnp.where(jj < 5, p1i + jj, p2i + jj)
                    v = plsc.load_gather(pcat_v, [src])
                    plsc.store_scatter(
                        rows_v[p], [t16, jj + WORD_DIM], v)
                return carry

            lax.fori_loop(0, CHUNK // LANES, grp, 0)

        stage_and_fire(0, 0)

        def body(i4, carry):
            for p in range(NBUF):
                j = NBUF * i4 + p
                q = (p + 1) % NBUF
                wait_gather(p)

                @pl.when(j + 1 < NCHUNK)
                def _():
                    @pl.when(j >= NBUF - 1)
                    def _():
                        wait_wb(q)

                    stage_and_fire(j + 1, q)

                pos_fill(p)
                pltpu.async_copy(
                    rows_v[p],
                    out_hbm.at[pl.ds(base_w + j * CHUNK, CHUNK), :],
                    wsem[p])
            return carry

        lax.fori_loop(0, NCHUNK // NBUF, body, 0)
        for p in range(NBUF):
            wait_wb(p)

    return k(word_pad, widx, p1idx, p2idx, p1t, p2t)


def kernel(word, pos1, pos2, word_table, pos1_table, pos2_table):
    word_pad = jnp.pad(word_table, ((0, 0), (0, PAD_DIM - WORD_DIM)))
    out_pad = _sc_embed(
        word_pad,
        word.reshape(-1),
        pos1.reshape(-1),
        pos2.reshape(-1),
        pos1_table.reshape(-1),
        pos2_table.reshape(-1),
    )
    return out_pad[:, :OUT_DIM].reshape(B, L, OUT_DIM)
